# baseline (speedup 1.0000x reference)
import functools

import jax
import jax.numpy as jnp
from jax import lax
from jax.experimental import pallas as pl
from jax.experimental.pallas import tpu as pltpu
from jax.experimental.pallas import tpu_sc as plsc

IOU_THRESHOLD = 0.5
MAX_DET = 300
PAD_DET = 304
BATCH = 8
NUM_BOXES = 20000
L = 16
NBLK = PAD_DET // L
NVEC = NUM_BOXES // L
INT32_MAX = 2147483647
RA_KEY = 0
RA_IDX = NUM_BOXES
RB_KEY = 2 * NUM_BOXES
RB_IDX = 3 * NUM_BOXES


def _nms_body(scores_hbm, boxes_hbm, cls_hbm,
              keep_out, score_out, bx1_out, by1_out, bx2_out, by2_out,
              cls_out, cnt_out,
              big_v, idx_s, scores_v, hist,
              kx1, ky1, kx2, ky2, karea,
              st_keep, st_safe, st_score, st_cls, st_cnt, cnt_smem):
    c = lax.axis_index("c")
    s = lax.axis_index("s")
    wid = s * 2 + c

    @pl.when(wid < BATCH)
    def _():
        b = wid
        lane = lax.iota(jnp.int32, L)
        ones = jnp.ones((L,), jnp.int32)
        zi = jnp.zeros((L,), jnp.int32)

        pltpu.sync_copy(scores_hbm.at[b], scores_v)

        def keys_body(v, carry):
            p = v * L
            sf = scores_v[pl.ds(p, L)]
            bits = plsc.bitcast(sf, jnp.int32)
            ku = jnp.where(bits < 0, ~bits, bits ^ jnp.int32(-2147483648))
            kd = ~ku
            big_v[pl.ds(RA_KEY + p, L)] = plsc.bitcast(kd, jnp.float32)
            iv = jnp.full((L,), p, jnp.int32) + lane
            big_v[pl.ds(RA_IDX + p, L)] = plsc.bitcast(iv, jnp.float32)
            return carry

        lax.fori_loop(0, NVEC, keys_body, jnp.int32(0))

        colbase = lane * jnp.int32(NVEC)

        def radix_pass(p, carry):
            even = (p % 2) == 0
            src_k = jnp.where(even, RA_KEY, RB_KEY)
            src_i = jnp.where(even, RA_IDX, RB_IDX)
            dst_k = jnp.where(even, RB_KEY, RA_KEY)
            dst_i = jnp.where(even, RB_IDX, RA_IDX)
            shiftv = jnp.full((L,), p * 8, jnp.int32)
            m255 = jnp.full((L,), 255, jnp.int32)
            sh4 = jnp.full((L,), 4, jnp.int32)

            def zero_body(i, cc):
                hist[pl.ds(i * L, L)] = zi
                return cc

            lax.fori_loop(0, 256, zero_body, jnp.int32(0))

            def hist_body(v, cc):
                kidx = colbase + v + src_k
                kf = plsc.load_gather(big_v, [kidx])
                k = plsc.bitcast(kf, jnp.int32)
                d = lax.shift_right_logical(k, shiftv) & m255
                addr = lax.shift_left(d, sh4) | lane
                plsc.addupdate_scatter(hist, [addr], ones)
                return cc

            lax.fori_loop(0, NVEC, hist_body, jnp.int32(0))

            def pfx_body(i, carry_in):
                vec = hist[pl.ds(i * L, L)]
                inc = plsc.cumsum(vec)
                hist[pl.ds(i * L, L)] = jnp.full((L,), carry_in) + inc - vec
                return carry_in + inc[L - 1]

            lax.fori_loop(0, 256, pfx_body, jnp.int32(0))

            def scat_body(v, cc):
                kidx = colbase + v + src_k
                kf = plsc.load_gather(big_v, [kidx])
                inf = plsc.load_gather(big_v, [kidx + NUM_BOXES])
                k = plsc.bitcast(kf, jnp.int32)
                d = lax.shift_right_logical(k, shiftv) & m255
                addr = lax.shift_left(d, sh4) | lane
                ofs = plsc.load_gather(hist, [addr])
                plsc.store_scatter(hist, [addr], ofs + 1)
                plsc.store_scatter(big_v, [ofs + dst_k], kf)
                plsc.store_scatter(big_v, [ofs + dst_i], inf)
                return cc

            lax.fori_loop(0, NVEC, scat_body, jnp.int32(0))
            return carry

        lax.fori_loop(0, 4, radix_pass, jnp.int32(0))

        def save_body(v, cc):
            p = v * L
            f = big_v[pl.ds(RA_IDX + p, L)]
            idx_s[pl.ds(p, L)] = plsc.bitcast(f, jnp.int32)
            return cc

        lax.fori_loop(0, NVEC, save_body, jnp.int32(0))

        pltpu.sync_copy(boxes_hbm.at[b], big_v)

        zf = jnp.zeros((L,), jnp.float32)
        neg1 = jnp.full((L,), -1, jnp.int32)
        for k in range(NBLK):
            sl = pl.ds(k * L, L)
            kx1[sl] = zf
            ky1[sl] = zf
            kx2[sl] = zf
            ky2[sl] = zf
            karea[sl] = zf
            st_keep[sl] = neg1
            st_safe[sl] = zi

        lane0 = lane == 0
        cnt_smem[0] = jnp.int32(0)

        def scan_blk(blk, carry):
            @pl.when(cnt_smem[0] < MAX_DET)
            def _():
                pos = blk * L
                idx16 = idx_s[pl.ds(pos, L)]
                i4 = idx16 * 4
                cx1 = plsc.load_gather(big_v, [i4])
                cy1 = plsc.load_gather(big_v, [i4 + 1])
                cx2 = plsc.load_gather(big_v, [i4 + 2])
                cy2 = plsc.load_gather(big_v, [i4 + 3])
                careas = (cx2 - cx1) * (cy2 - cy1)

                for j in range(L):
                    cnt = cnt_smem[0]
                    ax1 = jnp.full((L,), cx1[j])
                    ay1 = jnp.full((L,), cy1[j])
                    ax2 = jnp.full((L,), cx2[j])
                    ay2 = jnp.full((L,), cy2[j])
                    aar = jnp.full((L,), careas[j])

                    def iou_blk(k, acc):
                        sl = pl.ds(k * L, L)
                        w = jnp.maximum(
                            jnp.minimum(kx2[sl], ax2)
                            - jnp.maximum(kx1[sl], ax1), 0.0)
                        h = jnp.maximum(
                            jnp.minimum(ky2[sl], ay2)
                            - jnp.maximum(ky1[sl], ay1), 0.0)
                        inter = w * h
                        denom = karea[sl] + aar - inter
                        return jnp.maximum(acc, inter - IOU_THRESHOLD * denom)

                    metric = lax.fori_loop(0, NBLK, iou_blk,
                                           jnp.full((L,), -1.0, jnp.float32))
                    mmax = lax.reduce_max_p.bind(metric, axes=(0,))
                    keep_j = (cnt < MAX_DET) & (mmax <= 0.0)
                    m = lane0 & jnp.full((L,), keep_j)
                    idxv = jnp.full((L,), cnt, jnp.int32)
                    plsc.store_scatter(kx1, [idxv], ax1, mask=m)
                    plsc.store_scatter(ky1, [idxv], ay1, mask=m)
                    plsc.store_scatter(kx2, [idxv], ax2, mask=m)
                    plsc.store_scatter(ky2, [idxv], ay2, mask=m)
                    plsc.store_scatter(karea, [idxv], aar, mask=m)
                    oidx = jnp.full((L,), idx16[j], jnp.int32)
                    plsc.store_scatter(st_keep, [idxv], oidx, mask=m)
                    plsc.store_scatter(st_safe, [idxv], oidx, mask=m)
                    cnt_smem[0] = cnt + keep_j.astype(jnp.int32)

            return carry

        lax.fori_loop(0, NVEC, scan_blk, jnp.int32(0))
        cnt = cnt_smem[0]
        st_cnt[...] = jnp.full((L,), cnt, jnp.int32)

        pltpu.sync_copy(cls_hbm.at[b], idx_s)
        for k in range(NBLK):
            sl = pl.ds(k * L, L)
            kidx = st_safe[sl]
            raw = st_keep[sl]
            valid = raw >= 0
            sc = plsc.load_gather(scores_v, [kidx])
            cl = plsc.load_gather(idx_s, [kidx])
            st_score[sl] = jnp.where(valid, sc, 0.0)
            st_cls[sl] = jnp.where(valid, cl, INT32_MAX)

        pltpu.sync_copy(st_keep, keep_out.at[b])
        pltpu.sync_copy(st_score, score_out.at[b])
        pltpu.sync_copy(kx1, bx1_out.at[b])
        pltpu.sync_copy(ky1, by1_out.at[b])
        pltpu.sync_copy(kx2, bx2_out.at[b])
        pltpu.sync_copy(ky2, by2_out.at[b])
        pltpu.sync_copy(st_cls, cls_out.at[b])
        pltpu.sync_copy(st_cnt, cnt_out.at[b])


_sc_nms = functools.partial(
    pl.kernel,
    out_type=(
        jax.ShapeDtypeStruct((BATCH, PAD_DET), jnp.int32),
        jax.ShapeDtypeStruct((BATCH, PAD_DET), jnp.float32),
        jax.ShapeDtypeStruct((BATCH, PAD_DET), jnp.float32),
        jax.ShapeDtypeStruct((BATCH, PAD_DET), jnp.float32),
        jax.ShapeDtypeStruct((BATCH, PAD_DET), jnp.float32),
        jax.ShapeDtypeStruct((BATCH, PAD_DET), jnp.float32),
        jax.ShapeDtypeStruct((BATCH, PAD_DET), jnp.int32),
        jax.ShapeDtypeStruct((BATCH, L), jnp.int32),
    ),
    mesh=plsc.VectorSubcoreMesh(core_axis_name="c", subcore_axis_name="s"),
    compiler_params=pltpu.CompilerParams(needs_layout_passes=False),
    scratch_types=[
        pltpu.VMEM((4 * NUM_BOXES,), jnp.float32),
        pltpu.VMEM((NUM_BOXES,), jnp.int32),
        pltpu.VMEM((NUM_BOXES,), jnp.float32),
        pltpu.VMEM((4096,), jnp.int32),
        pltpu.VMEM((PAD_DET,), jnp.float32),
        pltpu.VMEM((PAD_DET,), jnp.float32),
        pltpu.VMEM((PAD_DET,), jnp.float32),
        pltpu.VMEM((PAD_DET,), jnp.float32),
        pltpu.VMEM((PAD_DET,), jnp.float32),
        pltpu.VMEM((PAD_DET,), jnp.int32),
        pltpu.VMEM((PAD_DET,), jnp.int32),
        pltpu.VMEM((PAD_DET,), jnp.float32),
        pltpu.VMEM((PAD_DET,), jnp.int32),
        pltpu.VMEM((L,), jnp.int32),
        pltpu.SMEM((1,), jnp.int32),
    ],
)(_nms_body)


def kernel(scores, boxes, classes):
    B, N = scores.shape
    boxes_flat = boxes.reshape(B, 4 * N)
    keep, osc, ox1, oy1, ox2, oy2, ocl, ocnt = _sc_nms(
        scores, boxes_flat, classes.astype(jnp.int32))
    out_boxes = jnp.stack([ox1, oy1, ox2, oy2], axis=-1)[:, :MAX_DET, :]
    return (
        keep[:, :MAX_DET],
        osc[:, :MAX_DET],
        out_boxes,
        ocl[:, :MAX_DET],
        ocnt[:, 0],
    )
